# baseline (speedup 1.0000x reference)
import functools

import jax
import jax.numpy as jnp
from jax import lax
from jax.experimental import pallas as pl
from jax.experimental.pallas import tpu as pltpu
from jax.experimental.pallas import tpu_sc as plsc

_NC = 2
_NS = 16
_NW = _NC * _NS
_L = 16

_D = 2048
_C = 2
_NBUF = 8


def _permute_rows(x2, perm32):
    R, D = x2.shape
    rows_per_w = R // _NW
    n_chunks = rows_per_w // _C
    assert n_chunks % _NBUF == 0
    groups = D // _L

    mesh = plsc.VectorSubcoreMesh(
        core_axis_name="c", subcore_axis_name="s",
        num_cores=_NC, num_subcores=_NS)

    @functools.partial(
        pl.kernel,
        mesh=mesh,
        out_type=jax.ShapeDtypeStruct((R, D), jnp.float32),
        scratch_types=[
            pltpu.VMEM((D,), jnp.int32),
            [pltpu.VMEM((_C, D), jnp.float32)] * _NBUF,
            [pltpu.VMEM((_C, D), jnp.float32)] * _NBUF,
            [pltpu.SemaphoreType.DMA] * _NBUF,
            [pltpu.SemaphoreType.DMA] * _NBUF,
        ],
        compiler_params=pltpu.CompilerParams(needs_layout_passes=False),
    )
    def k(x_hbm, perm_hbm, out_hbm, idx_v, ins, outs, sis, sos):
        wid = lax.axis_index("s") * _NC + lax.axis_index("c")
        base = wid * rows_per_w
        pltpu.sync_copy(perm_hbm, idx_v)

        def start_in(c, b):
            pltpu.async_copy(x_hbm.at[pl.ds(base + c * _C, _C)], ins[b], sis[b])

        def wait_in(b):
            pltpu.make_async_copy(x_hbm.at[pl.ds(0, _C)], ins[b], sis[b]).wait()

        def start_out(c, b):
            pltpu.async_copy(outs[b], out_hbm.at[pl.ds(base + c * _C, _C)], sos[b])

        def wait_out(b):
            pltpu.make_async_copy(outs[b], out_hbm.at[pl.ds(0, _C)], sos[b]).wait()

        for b in range(_NBUF - 1):
            start_in(b, b)

        @pl.loop(0, n_chunks, step=_NBUF)
        def _ring(c0):
            for b in range(_NBUF):
                c = c0 + b

                @pl.when(c + _NBUF - 1 < n_chunks)
                def _prefetch():
                    start_in(c + _NBUF - 1, (b + _NBUF - 1) % _NBUF)

                wait_in(b)

                @pl.when(c >= _NBUF)
                def _drain():
                    wait_out(b)

                @plsc.parallel_loop(0, groups, unroll=4)
                def _group(j):
                    col = j * _L
                    idx = idx_v[pl.ds(col, _L)]
                    for r in range(_C):
                        row = jnp.full((_L,), r, dtype=jnp.int32)
                        vals = plsc.load_gather(ins[b], [row, idx])
                        outs[b][r, pl.ds(col, _L)] = vals

                start_out(c, b)

        for b in range(_NBUF):
            wait_out(b)

    return k(x2, perm32)


def kernel(x, perm):
    B, S, D = x.shape
    x2 = x.reshape(B * S, D)
    out2 = _permute_rows(x2, perm.astype(jnp.int32))
    out = out2.reshape(B, S, D)
    log_det = jnp.zeros((B, S), dtype=x.dtype)
    return (out, log_det)


# speedup vs baseline: 1.0067x; 1.0067x over previous
import functools

import jax
import jax.numpy as jnp
from jax import lax
from jax.experimental import pallas as pl
from jax.experimental.pallas import tpu as pltpu
from jax.experimental.pallas import tpu_sc as plsc

_NC = 2
_NS = 16
_NW = _NC * _NS
_L = 16

_D = 2048
_C = 8
_NBUF = 3


def _permute_rows(x2, perm32):
    R, D = x2.shape
    rows_per_w = R // _NW
    n_chunks = rows_per_w // _C
    n_main = (n_chunks // _NBUF) * _NBUF
    groups = D // _L

    mesh = plsc.VectorSubcoreMesh(
        core_axis_name="c", subcore_axis_name="s",
        num_cores=_NC, num_subcores=_NS)

    @functools.partial(
        pl.kernel,
        mesh=mesh,
        out_type=jax.ShapeDtypeStruct((R, D), jnp.float32),
        scratch_types=[
            pltpu.VMEM((D,), jnp.int32),
            [pltpu.VMEM((_C, D), jnp.float32)] * _NBUF,
            [pltpu.VMEM((_C, D), jnp.float32)] * _NBUF,
            [pltpu.SemaphoreType.DMA] * _NBUF,
            [pltpu.SemaphoreType.DMA] * _NBUF,
        ],
        compiler_params=pltpu.CompilerParams(needs_layout_passes=False),
    )
    def k(x_hbm, perm_hbm, out_hbm, idx_v, ins, outs, sis, sos):
        wid = lax.axis_index("s") * _NC + lax.axis_index("c")
        base = wid * rows_per_w
        pltpu.sync_copy(perm_hbm, idx_v)

        def start_in(c, b):
            pltpu.async_copy(x_hbm.at[pl.ds(base + c * _C, _C)], ins[b], sis[b])

        def wait_in(b):
            pltpu.make_async_copy(x_hbm.at[pl.ds(0, _C)], ins[b], sis[b]).wait()

        def start_out(c, b):
            pltpu.async_copy(outs[b], out_hbm.at[pl.ds(base + c * _C, _C)], sos[b])

        def wait_out(b):
            pltpu.make_async_copy(outs[b], out_hbm.at[pl.ds(0, _C)], sos[b]).wait()

        def compute(b):
            @plsc.parallel_loop(0, groups, unroll=4)
            def _group(j):
                col = j * _L
                idx = idx_v[pl.ds(col, _L)]
                for r in range(_C):
                    row = jnp.full((_L,), r, dtype=jnp.int32)
                    vals = plsc.load_gather(ins[b], [row, idx])
                    outs[b][r, pl.ds(col, _L)] = vals

        for b in range(_NBUF - 1):
            start_in(b, b)

        @pl.loop(0, n_main, step=_NBUF)
        def _ring(c0):
            for b in range(_NBUF):
                c = c0 + b

                @pl.when(c + _NBUF - 1 < n_chunks)
                def _prefetch():
                    start_in(c + _NBUF - 1, (b + _NBUF - 1) % _NBUF)

                wait_in(b)

                @pl.when(c >= _NBUF)
                def _drain():
                    wait_out(b)

                compute(b)
                start_out(c, b)

        for c in range(n_main, n_chunks):
            b = c % _NBUF
            if c + _NBUF - 1 < n_chunks:
                start_in(c + _NBUF - 1, (c + _NBUF - 1) % _NBUF)
            wait_in(b)
            if c >= _NBUF:
                wait_out(b)
            compute(b)
            start_out(c, b)

        for b in range(min(_NBUF, n_chunks)):
            wait_out(b)

    return k(x2, perm32)


def kernel(x, perm):
    B, S, D = x.shape
    x2 = x.reshape(B * S, D)
    out2 = _permute_rows(x2, perm.astype(jnp.int32))
    out = out2.reshape(B, S, D)
    log_det = jnp.zeros((B, S), dtype=x.dtype)
    return (out, log_det)
